# baseline (speedup 1.0000x reference)
import math

import jax
import jax.numpy as jnp
from jax.experimental import pallas as pl
from jax.experimental.pallas import tpu as pltpu

D = 2048
SEQ = 2048
B = 2
BM = 256
NI = SEQ // BM
SCALE = 1.0 / math.sqrt(D)
BF = jnp.bfloat16

_NT = (((1,), (1,)), ((), ()))


def _body(x1_ref, x2_ref, wqf_ref, wkf_ref, bq_ref, bk_ref, aff_ref, pen_ref,
          wq_s, wk_s, k_s, q_s, s_s):
    p = pl.program_id(0)
    j = pl.program_id(1)

    @pl.when((p == 0) & (j < NI))
    def _cast_wk():
        wk_s[pl.ds(j * BM, BM), :] = wkf_ref[...].astype(BF)

    @pl.when(((p == 1) | (p == 3)) & (j < NI))
    def _kproj():
        x = x1_ref[0].astype(BF)
        kt = jax.lax.dot_general(x, wk_s[...], _NT,
                                 preferred_element_type=jnp.float32)
        k_s[pl.ds(j * BM, BM), :] = (kt + bk_ref[...]).astype(BF)

    @pl.when((p == 1) & (j < NI))
    def _cast_wq():
        wq_s[pl.ds(j * BM, BM), :] = wqf_ref[...].astype(BF)

    @pl.when((p == 2) | (p == 4))
    def _scores():
        @pl.when(j < NI)
        def _stage_q():
            xq = x2_ref[0].astype(BF)
            qt = jax.lax.dot_general(xq, wq_s[...], _NT,
                                     preferred_element_type=jnp.float32)
            q_s[pl.ds(j % 2, 1)] = ((qt + bq_ref[...]) * SCALE).astype(BF)[None]

        @pl.when((j >= 1) & (j <= NI))
        def _stage_s():
            qq = q_s[pl.ds((j - 1) % 2, 1)][0]
            s = jax.lax.dot_general(qq, k_s[...], _NT,
                                    preferred_element_type=jnp.float32)
            s_s[pl.ds((j - 1) % 2, 1)] = s.astype(BF)[None]

        @pl.when(j >= 2)
        def _stage_epi():
            t = jnp.tanh(s_s[pl.ds(j % 2, 1)][0].astype(jnp.float32))
            aff_ref[0] = jnp.maximum(t, 0.0)
            pen_ref[0] = jnp.minimum(t, 0.0)


def kernel(x, Wq, bq, Wk, bk):
    bq2 = bq.reshape(1, D)
    bk2 = bk.reshape(1, D)

    x1_map = lambda p, j: (
        jnp.where(p == 0, 0, (p - 1) // 2),
        jnp.where((p == 1) | (p == 3), jnp.minimum(j, NI - 1),
                  jnp.where(p == 0, 0, NI - 1)), 0)
    x2_map = lambda p, j: (
        jnp.where(p <= 2, 0, 1),
        jnp.where((p == 2) | (p == 4), jnp.minimum(j, NI - 1), 0), 0)
    wq_map = lambda p, j: (
        jnp.where(p == 0, 0,
                  jnp.where(p == 1, jnp.minimum(j, NI - 1), NI - 1)), 0)
    wk_map = lambda p, j: (
        jnp.where(p == 0, jnp.minimum(j, NI - 1), NI - 1), 0)
    out_map = lambda p, j: (
        jnp.where(p < 4, 0, 1),
        jnp.where((p == 2) | (p == 4), jnp.clip(j - 2, 0, NI - 1),
                  jnp.where(p == 3, NI - 1, 0)), 0)
    const = lambda p, j: (0, 0)

    aff, pen = pl.pallas_call(
        _body,
        grid=(5, NI + 2),
        in_specs=[
            pl.BlockSpec((1, BM, D), x1_map),
            pl.BlockSpec((1, BM, D), x2_map),
            pl.BlockSpec((BM, D), wq_map),
            pl.BlockSpec((BM, D), wk_map),
            pl.BlockSpec((1, D), const),
            pl.BlockSpec((1, D), const),
        ],
        out_specs=[
            pl.BlockSpec((1, BM, SEQ), out_map),
            pl.BlockSpec((1, BM, SEQ), out_map),
        ],
        out_shape=[
            jax.ShapeDtypeStruct((B, SEQ, SEQ), jnp.float32),
            jax.ShapeDtypeStruct((B, SEQ, SEQ), jnp.float32),
        ],
        scratch_shapes=[
            pltpu.VMEM((D, D), BF),
            pltpu.VMEM((D, D), BF),
            pltpu.VMEM((SEQ, D), BF),
            pltpu.VMEM((2, BM, D), BF),
            pltpu.VMEM((2, BM, SEQ), BF),
        ],
        compiler_params=pltpu.CompilerParams(
            dimension_semantics=("arbitrary", "arbitrary")),
    )(x, x, Wq, Wk, bq2, bk2)

    return aff, pen


# speedup vs baseline: 1.1379x; 1.1379x over previous
import math

import jax
import jax.numpy as jnp
from jax.experimental import pallas as pl
from jax.experimental.pallas import tpu as pltpu

D = 2048
SEQ = 2048
B = 2
BM = 256
NI = SEQ // BM
SCALE = 1.0 / math.sqrt(D)
BF = jnp.bfloat16

_NT = (((1,), (1,)), ((), ()))


def _body(x_ref, wqf_ref, wkf_ref, bq_ref, bk_ref, aff_ref, pen_ref,
          wq_s, wk_s, k_s):
    p = pl.program_id(0)
    i = pl.program_id(1)

    @pl.when(p == 0)
    def _cast_wk():
        wk_s[pl.ds(i * BM, BM), :] = wkf_ref[...].astype(BF)

    @pl.when(p == 1)
    def _cast_wq():
        wq_s[pl.ds(i * BM, BM), :] = wqf_ref[...].astype(BF)

    @pl.when((p == 1) | (p == 3))
    def _kproj():
        x = x_ref[0].astype(BF)
        kt = jax.lax.dot_general(x, wk_s[...], _NT,
                                 preferred_element_type=jnp.float32)
        k_s[pl.ds(i * BM, BM), :] = (kt + bk_ref[...]).astype(BF)

    @pl.when((p == 2) | (p == 4))
    def _scores():
        x = x_ref[0].astype(BF)
        qt = jax.lax.dot_general(x, wq_s[...], _NT,
                                 preferred_element_type=jnp.float32)
        q = (qt + bq_ref[...]).astype(BF)
        s = jax.lax.dot_general(q, k_s[...], _NT,
                                preferred_element_type=jnp.float32)
        t = jnp.tanh(s * SCALE)
        aff_ref[0] = jnp.maximum(t, 0.0)
        pen_ref[0] = jnp.minimum(t, 0.0)


def kernel(x, Wq, bq, Wk, bk):
    bq2 = bq.reshape(1, D)
    bk2 = bk.reshape(1, D)

    x_map = lambda p, i: (jnp.where(p == 0, 0, (p - 1) // 2),
                          jnp.where(p == 0, 0, i), 0)
    wq_map = lambda p, i: (jnp.where(p <= 1, jnp.where(p == 1, i, 0), NI - 1), 0)
    wk_map = lambda p, i: (jnp.where(p == 0, i, NI - 1), 0)
    out_map = lambda p, i: (
        jnp.where(p < 4, 0, 1),
        jnp.where((p == 2) | (p == 4), i, jnp.where(p < 2, 0, NI - 1)), 0)
    const = lambda p, i: (0, 0)

    aff, pen = pl.pallas_call(
        _body,
        grid=(5, NI),
        in_specs=[
            pl.BlockSpec((1, BM, D), x_map),
            pl.BlockSpec((BM, D), wq_map),
            pl.BlockSpec((BM, D), wk_map),
            pl.BlockSpec((1, D), const),
            pl.BlockSpec((1, D), const),
        ],
        out_specs=[
            pl.BlockSpec((1, BM, SEQ), out_map),
            pl.BlockSpec((1, BM, SEQ), out_map),
        ],
        out_shape=[
            jax.ShapeDtypeStruct((B, SEQ, SEQ), jnp.float32),
            jax.ShapeDtypeStruct((B, SEQ, SEQ), jnp.float32),
        ],
        scratch_shapes=[
            pltpu.VMEM((D, D), BF),
            pltpu.VMEM((D, D), BF),
            pltpu.VMEM((SEQ, D), BF),
        ],
        compiler_params=pltpu.CompilerParams(
            dimension_semantics=("arbitrary", "arbitrary")),
    )(x, Wq, Wk, bq2, bk2)

    return aff, pen
